# Initial kernel scaffold; baseline (speedup 1.0000x reference)
import functools

import jax
import jax.numpy as jnp
from jax import lax
from jax.experimental import pallas as pl
from jax.experimental.pallas import tpu as pltpu
from jax.experimental.pallas import tpu_sc as plsc

N = 10000
E = 320000
D = 128
ED = 16
H = 128
L = 3

N_PAD = 10240
E_PAD = 327680
NW = 32
EPW = E_PAD // NW
IDXROWS = EPW // 128
CHUNK = 512
NCHUNK = EPW // CHUNK
RPC = CHUNK // 128
TILE_ROWS = N_PAD // 16

_MESH = plsc.VectorSubcoreMesh(core_axis_name="c", subcore_axis_name="s")


def _gather_body(tbl_hbm, gidx_hbm, out_hbm, idx_v, rows_v, sem):
    wid = lax.axis_index("s") * 2 + lax.axis_index("c")
    pltpu.sync_copy(gidx_hbm.at[wid], idx_v)
    base = wid * EPW

    @pl.loop(0, NCHUNK)
    def _chunk(c):
        r0 = c * RPC
        cps = [
            pltpu.async_copy(
                tbl_hbm.at[idx_v.at[0, r0 + r]],
                rows_v.at[pl.ds(r * 128, 128)],
                sem,
            )
            for r in range(RPC)
        ]
        for cp in cps:
            cp.wait()
        cps = [
            pltpu.async_copy(
                tbl_hbm.at[idx_v.at[1, r0 + r]],
                rows_v.at[pl.ds(r * 128, 128)],
                sem,
                add=True,
            )
            for r in range(RPC)
        ]
        for cp in cps:
            cp.wait()
        pltpu.sync_copy(rows_v, out_hbm.at[pl.ds(base + c * CHUNK, CHUNK)])


_sc_gather = pl.kernel(
    _gather_body,
    out_type=jax.ShapeDtypeStruct((E_PAD, H), jnp.float32),
    mesh=_MESH,
    scratch_types=[
        pltpu.VMEM((2, IDXROWS, 128), jnp.int32),
        pltpu.VMEM((CHUNK, H), jnp.float32),
        pltpu.SemaphoreType.DMA,
    ],
)


def _scatter_body(m_hbm, sidx_hbm, zrow_hbm, out_hbm, idx_v, rows_v, shared):
    cid = lax.axis_index("c")
    sid = lax.axis_index("s")
    wid = sid * 2 + cid
    pltpu.sync_copy(sidx_hbm.at[wid], idx_v)
    pltpu.sync_copy(zrow_hbm, shared.at[pl.ds(sid * TILE_ROWS, TILE_ROWS)])
    plsc.subcore_barrier()
    base = wid * EPW

    @pl.loop(0, NCHUNK)
    def _chunk(c):
        pltpu.sync_copy(m_hbm.at[pl.ds(base + c * CHUNK, CHUNK)], rows_v)
        r0 = c * RPC
        for r in range(RPC):
            pltpu.sync_copy(
                rows_v.at[pl.ds(r * 128, 128)],
                shared.at[idx_v.at[r0 + r]],
                add=True,
            )

    plsc.subcore_barrier()
    pltpu.sync_copy(
        shared.at[pl.ds(sid * TILE_ROWS, TILE_ROWS)],
        out_hbm.at[cid, pl.ds(sid * TILE_ROWS, TILE_ROWS)],
    )


_sc_scatter = pl.kernel(
    _scatter_body,
    out_type=jax.ShapeDtypeStruct((2, N_PAD, H), jnp.float32),
    mesh=_MESH,
    scratch_types=[
        pltpu.VMEM((IDXROWS, 128), jnp.int32),
        pltpu.VMEM((CHUNK, H), jnp.float32),
        pltpu.VMEM_SHARED((N_PAD, H), jnp.float32),
    ],
)


_BN = 1024


def _proj_body(x_ref, w_ref, b_ref, out_ref):
    out_ref[...] = (
        jnp.dot(x_ref[...], w_ref[...], preferred_element_type=jnp.float32)
        + b_ref[...]
    )


def _tc_proj(x, w, b):
    return pl.pallas_call(
        _proj_body,
        grid=(N_PAD // _BN,),
        in_specs=[
            pl.BlockSpec((_BN, D), lambda i: (i, 0)),
            pl.BlockSpec((D, H), lambda i: (0, 0)),
            pl.BlockSpec((1, H), lambda i: (0, 0)),
        ],
        out_specs=pl.BlockSpec((_BN, H), lambda i: (i, 0)),
        out_shape=jax.ShapeDtypeStruct((N_PAD, H), jnp.float32),
    )(x, w, b)


def _ab_body(h_ref, wa_ref, wb_ref, b1_ref, out_ref):
    hb = h_ref[...]
    out_ref[0] = (
        jnp.dot(hb, wa_ref[...], preferred_element_type=jnp.float32) + b1_ref[...]
    )
    out_ref[1] = jnp.dot(hb, wb_ref[...], preferred_element_type=jnp.float32)


def _tc_ab(h, wa, wb, b1):
    return pl.pallas_call(
        _ab_body,
        grid=(N_PAD // _BN,),
        in_specs=[
            pl.BlockSpec((_BN, H), lambda i: (i, 0)),
            pl.BlockSpec((H, H), lambda i: (0, 0)),
            pl.BlockSpec((H, H), lambda i: (0, 0)),
            pl.BlockSpec((1, H), lambda i: (0, 0)),
        ],
        out_specs=pl.BlockSpec((2, _BN, H), lambda i: (0, i, 0)),
        out_shape=jax.ShapeDtypeStruct((2, N_PAD, H), jnp.float32),
    )(h, wa, wb, b1)


def _edge_body(g_ref, ea_ref, wc_ref, w2_ref, b2_ref, out_ref):
    m1 = jnp.maximum(
        g_ref[...]
        + jnp.dot(ea_ref[...], wc_ref[...], preferred_element_type=jnp.float32),
        0.0,
    )
    out_ref[...] = jnp.maximum(
        jnp.dot(m1, w2_ref[...], preferred_element_type=jnp.float32) + b2_ref[...],
        0.0,
    )


def _tc_edge(gsum, ea, wc, w2, b2):
    return pl.pallas_call(
        _edge_body,
        grid=(E_PAD // _BN,),
        in_specs=[
            pl.BlockSpec((_BN, H), lambda i: (i, 0)),
            pl.BlockSpec((_BN, ED), lambda i: (i, 0)),
            pl.BlockSpec((ED, H), lambda i: (0, 0)),
            pl.BlockSpec((H, H), lambda i: (0, 0)),
            pl.BlockSpec((1, H), lambda i: (0, 0)),
        ],
        out_specs=pl.BlockSpec((_BN, H), lambda i: (i, 0)),
        out_shape=jax.ShapeDtypeStruct((E_PAD, H), jnp.float32),
    )(gsum, ea, wc, w2, b2)


def _upd_body(h_ref, a0_ref, a1_ref, wh_ref, wa_ref, bu_ref, g_ref, b_ref, out_ref):
    hb = h_ref[...]
    agg = a0_ref[0] + a1_ref[0]
    o = (
        jnp.dot(hb, wh_ref[...], preferred_element_type=jnp.float32)
        + jnp.dot(agg, wa_ref[...], preferred_element_type=jnp.float32)
        + bu_ref[...]
    )
    o = jnp.maximum(o, 0.0) + hb
    mu = jnp.mean(o, axis=1, keepdims=True)
    var = jnp.mean((o - mu) * (o - mu), axis=1, keepdims=True)
    out_ref[...] = (o - mu) * lax.rsqrt(var + 1e-5) * g_ref[...] + b_ref[...]


def _tc_upd(h, scat, wh, wa, bu, g, b):
    return pl.pallas_call(
        _upd_body,
        grid=(N_PAD // _BN,),
        in_specs=[
            pl.BlockSpec((_BN, H), lambda i: (i, 0)),
            pl.BlockSpec((1, _BN, H), lambda i: (0, i, 0)),
            pl.BlockSpec((1, _BN, H), lambda i: (1, i, 0)),
            pl.BlockSpec((H, H), lambda i: (0, 0)),
            pl.BlockSpec((H, H), lambda i: (0, 0)),
            pl.BlockSpec((1, H), lambda i: (0, 0)),
            pl.BlockSpec((1, H), lambda i: (0, 0)),
            pl.BlockSpec((1, H), lambda i: (0, 0)),
        ],
        out_specs=pl.BlockSpec((_BN, H), lambda i: (i, 0)),
        out_shape=jax.ShapeDtypeStruct((N_PAD, H), jnp.float32),
    )(h, scat, scat, wh, wa, bu, g, b)


def _final_body(h_ref, g_ref, b_ref, out_ref):
    i = pl.program_id(0)
    hb = h_ref[...]
    mu = jnp.mean(hb, axis=1, keepdims=True)
    var = jnp.mean((hb - mu) * (hb - mu), axis=1, keepdims=True)
    y = (hb - mu) * lax.rsqrt(var + 1e-5) * g_ref[...] + b_ref[...]
    rows = i * _BN + lax.broadcasted_iota(jnp.int32, (_BN, 1), 0)
    y = jnp.where(rows < N, y, 0.0)
    part = jnp.sum(y, axis=0, keepdims=True)

    @pl.when(i == 0)
    def _():
        out_ref[...] = jnp.zeros_like(out_ref)

    out_ref[...] += part

    @pl.when(i == N_PAD // _BN - 1)
    def _():
        out_ref[...] *= 1.0 / N


def _tc_final(h, g, b):
    return pl.pallas_call(
        _final_body,
        grid=(N_PAD // _BN,),
        in_specs=[
            pl.BlockSpec((_BN, H), lambda i: (i, 0)),
            pl.BlockSpec((1, H), lambda i: (0, 0)),
            pl.BlockSpec((1, H), lambda i: (0, 0)),
        ],
        out_specs=pl.BlockSpec((1, H), lambda i: (0, 0)),
        out_shape=jax.ShapeDtypeStruct((1, H), jnp.float32),
    )(h, g, b)


def kernel(x, edge_index, edge_attr, proj_W, proj_b, msg_W1, msg_b1, msg_W2,
           msg_b2, upd_W, upd_b, ln_g, ln_b, out_g, out_b):
    f32 = jnp.float32
    i_idx = edge_index[0].astype(jnp.int32)
    j_idx = edge_index[1].astype(jnp.int32)
    pad_e = E_PAD - E

    gi = jnp.concatenate([i_idx, jnp.zeros((pad_e,), jnp.int32)])
    gj = jnp.concatenate([j_idx + N_PAD, jnp.full((pad_e,), N_PAD, jnp.int32)])
    gidx = jnp.stack(
        [gi.reshape(NW, IDXROWS, 128), gj.reshape(NW, IDXROWS, 128)], axis=1
    )
    sidx = jnp.concatenate(
        [i_idx, jnp.full((pad_e,), N, jnp.int32)]
    ).reshape(NW, IDXROWS, 128)

    x_pad = jnp.pad(x, ((0, N_PAD - N), (0, 0)))
    ea_pad = jnp.pad(edge_attr, ((0, pad_e), (0, 0)))
    zrow = jnp.zeros((TILE_ROWS, H), f32)

    h = _tc_proj(x_pad, proj_W, proj_b.reshape(1, H))
    for l in range(L):
        w1 = msg_W1[l]
        tbl2 = _tc_ab(h, w1[:H], w1[H : 2 * H], msg_b1[l].reshape(1, H))
        gsum = _sc_gather(tbl2.reshape(2 * N_PAD, H), gidx)
        m = _tc_edge(gsum, ea_pad, w1[2 * H :], msg_W2[l], msg_b2[l].reshape(1, H))
        scat = _sc_scatter(m, sidx, zrow)
        h = _tc_upd(
            h,
            scat,
            upd_W[l][:H],
            upd_W[l][H:],
            upd_b[l].reshape(1, H),
            ln_g[l].reshape(1, H),
            ln_b[l].reshape(1, H),
        )
    return _tc_final(h, out_g.reshape(1, H), out_b.reshape(1, H))


# speedup vs baseline: 1.5939x; 1.5939x over previous
import functools

import jax
import jax.numpy as jnp
from jax import lax
from jax.experimental import pallas as pl
from jax.experimental.pallas import tpu as pltpu
from jax.experimental.pallas import tpu_sc as plsc

N = 10000
E = 320000
D = 128
ED = 16
H = 128
L = 3

N_PAD = 10240
E_PAD = 327680
NW = 32
EPW = E_PAD // NW
IDXROWS = EPW // 128
CHUNK = 512
NCHUNK = EPW // CHUNK
RPC = CHUNK // 128
SCHUNK = 256
SNCHUNK = EPW // SCHUNK
SRPC = SCHUNK // 128
TILE_ROWS = N_PAD // 16

@functools.lru_cache(maxsize=None)
def _mesh():
    return plsc.VectorSubcoreMesh(
        core_axis_name="c", subcore_axis_name="s", num_cores=2, num_subcores=16
    )


def _gather_body(tbl_hbm, gidx_hbm, out_hbm, idx_v, rows_v, sem):
    wid = lax.axis_index("s") * 2 + lax.axis_index("c")
    pltpu.sync_copy(gidx_hbm.at[wid], idx_v)
    base = wid * EPW

    @pl.loop(0, NCHUNK)
    def _chunk(c):
        r0 = c * RPC
        cps = [
            pltpu.async_copy(
                tbl_hbm.at[idx_v.at[0, r0 + r]],
                rows_v.at[pl.ds(r * 128, 128)],
                sem,
            )
            for r in range(RPC)
        ]
        for cp in cps:
            cp.wait()
        cps = [
            pltpu.async_copy(
                tbl_hbm.at[idx_v.at[1, r0 + r]],
                rows_v.at[pl.ds(r * 128, 128)],
                sem,
                add=True,
            )
            for r in range(RPC)
        ]
        for cp in cps:
            cp.wait()
        pltpu.sync_copy(rows_v, out_hbm.at[pl.ds(base + c * CHUNK, CHUNK)])


@functools.lru_cache(maxsize=None)
def _sc_gather_kernel():
    return pl.kernel(
        _gather_body,
        out_type=jax.ShapeDtypeStruct((E_PAD, H), jnp.float32),
        mesh=_mesh(),
        scratch_types=[
            pltpu.VMEM((2, IDXROWS, 128), jnp.int32),
            pltpu.VMEM((CHUNK, H), jnp.float32),
            pltpu.SemaphoreType.DMA,
        ],
    )


def _scatter_body(m_hbm, sidx_hbm, zrow_hbm, out_hbm, idx_v, rows_v, shared):
    cid = lax.axis_index("c")
    sid = lax.axis_index("s")
    wid = sid * 2 + cid
    pltpu.sync_copy(sidx_hbm.at[wid], idx_v)
    pltpu.sync_copy(zrow_hbm, shared.at[pl.ds(sid * TILE_ROWS, TILE_ROWS)])
    plsc.subcore_barrier()
    base = wid * EPW

    @pl.loop(0, SNCHUNK)
    def _chunk(c):
        pltpu.sync_copy(m_hbm.at[pl.ds(base + c * SCHUNK, SCHUNK)], rows_v)
        r0 = c * SRPC
        for r in range(SRPC):
            pltpu.sync_copy(
                rows_v.at[pl.ds(r * 128, 128)],
                shared.at[idx_v.at[r0 + r]],
                add=True,
            )

    plsc.subcore_barrier()
    pltpu.sync_copy(
        shared.at[pl.ds(sid * TILE_ROWS, TILE_ROWS)],
        out_hbm.at[cid, pl.ds(sid * TILE_ROWS, TILE_ROWS)],
    )


@functools.lru_cache(maxsize=None)
def _sc_scatter_kernel():
    return pl.kernel(
        _scatter_body,
        out_type=jax.ShapeDtypeStruct((2, N_PAD, H), jnp.float32),
        mesh=_mesh(),
        scratch_types=[
            pltpu.VMEM((IDXROWS, 128), jnp.int32),
            pltpu.VMEM((SCHUNK, H), jnp.float32),
            pltpu.VMEM_SHARED((N_PAD, H), jnp.float32),
        ],
    )


_BN = 1024


def _proj_body(x_ref, w_ref, b_ref, out_ref):
    out_ref[...] = (
        jnp.dot(x_ref[...], w_ref[...], preferred_element_type=jnp.float32)
        + b_ref[...]
    )


def _tc_proj(x, w, b):
    return pl.pallas_call(
        _proj_body,
        grid=(N_PAD // _BN,),
        in_specs=[
            pl.BlockSpec((_BN, D), lambda i: (i, 0)),
            pl.BlockSpec((D, H), lambda i: (0, 0)),
            pl.BlockSpec((1, H), lambda i: (0, 0)),
        ],
        out_specs=pl.BlockSpec((_BN, H), lambda i: (i, 0)),
        out_shape=jax.ShapeDtypeStruct((N_PAD, H), jnp.float32),
    )(x, w, b)


def _ab_body(h_ref, wa_ref, wb_ref, b1_ref, out_ref):
    hb = h_ref[...]
    out_ref[0] = (
        jnp.dot(hb, wa_ref[...], preferred_element_type=jnp.float32) + b1_ref[...]
    )
    out_ref[1] = jnp.dot(hb, wb_ref[...], preferred_element_type=jnp.float32)


def _tc_ab(h, wa, wb, b1):
    return pl.pallas_call(
        _ab_body,
        grid=(N_PAD // _BN,),
        in_specs=[
            pl.BlockSpec((_BN, H), lambda i: (i, 0)),
            pl.BlockSpec((H, H), lambda i: (0, 0)),
            pl.BlockSpec((H, H), lambda i: (0, 0)),
            pl.BlockSpec((1, H), lambda i: (0, 0)),
        ],
        out_specs=pl.BlockSpec((2, _BN, H), lambda i: (0, i, 0)),
        out_shape=jax.ShapeDtypeStruct((2, N_PAD, H), jnp.float32),
    )(h, wa, wb, b1)


def _edge_body(g_ref, ea_ref, wc_ref, w2_ref, b2_ref, out_ref):
    m1 = jnp.maximum(
        g_ref[...]
        + jnp.dot(ea_ref[...], wc_ref[...], preferred_element_type=jnp.float32),
        0.0,
    )
    out_ref[...] = jnp.maximum(
        jnp.dot(m1, w2_ref[...], preferred_element_type=jnp.float32) + b2_ref[...],
        0.0,
    )


def _tc_edge(gsum, ea, wc, w2, b2):
    return pl.pallas_call(
        _edge_body,
        grid=(E_PAD // _BN,),
        in_specs=[
            pl.BlockSpec((_BN, H), lambda i: (i, 0)),
            pl.BlockSpec((_BN, ED), lambda i: (i, 0)),
            pl.BlockSpec((ED, H), lambda i: (0, 0)),
            pl.BlockSpec((H, H), lambda i: (0, 0)),
            pl.BlockSpec((1, H), lambda i: (0, 0)),
        ],
        out_specs=pl.BlockSpec((_BN, H), lambda i: (i, 0)),
        out_shape=jax.ShapeDtypeStruct((E_PAD, H), jnp.float32),
    )(gsum, ea, wc, w2, b2)


def _upd_body(h_ref, a0_ref, a1_ref, wh_ref, wa_ref, bu_ref, g_ref, b_ref, out_ref):
    hb = h_ref[...]
    agg = a0_ref[0] + a1_ref[0]
    o = (
        jnp.dot(hb, wh_ref[...], preferred_element_type=jnp.float32)
        + jnp.dot(agg, wa_ref[...], preferred_element_type=jnp.float32)
        + bu_ref[...]
    )
    o = jnp.maximum(o, 0.0) + hb
    mu = jnp.mean(o, axis=1, keepdims=True)
    var = jnp.mean((o - mu) * (o - mu), axis=1, keepdims=True)
    out_ref[...] = (o - mu) * lax.rsqrt(var + 1e-5) * g_ref[...] + b_ref[...]


def _tc_upd(h, scat, wh, wa, bu, g, b):
    return pl.pallas_call(
        _upd_body,
        grid=(N_PAD // _BN,),
        in_specs=[
            pl.BlockSpec((_BN, H), lambda i: (i, 0)),
            pl.BlockSpec((1, _BN, H), lambda i: (0, i, 0)),
            pl.BlockSpec((1, _BN, H), lambda i: (1, i, 0)),
            pl.BlockSpec((H, H), lambda i: (0, 0)),
            pl.BlockSpec((H, H), lambda i: (0, 0)),
            pl.BlockSpec((1, H), lambda i: (0, 0)),
            pl.BlockSpec((1, H), lambda i: (0, 0)),
            pl.BlockSpec((1, H), lambda i: (0, 0)),
        ],
        out_specs=pl.BlockSpec((_BN, H), lambda i: (i, 0)),
        out_shape=jax.ShapeDtypeStruct((N_PAD, H), jnp.float32),
    )(h, scat, scat, wh, wa, bu, g, b)


def _final_body(h_ref, g_ref, b_ref, out_ref):
    i = pl.program_id(0)
    hb = h_ref[...]
    mu = jnp.mean(hb, axis=1, keepdims=True)
    var = jnp.mean((hb - mu) * (hb - mu), axis=1, keepdims=True)
    y = (hb - mu) * lax.rsqrt(var + 1e-5) * g_ref[...] + b_ref[...]
    rows = i * _BN + lax.broadcasted_iota(jnp.int32, (_BN, 1), 0)
    y = jnp.where(rows < N, y, 0.0)
    part = jnp.sum(y, axis=0, keepdims=True)

    @pl.when(i == 0)
    def _():
        out_ref[...] = jnp.zeros_like(out_ref)

    out_ref[...] += part

    @pl.when(i == N_PAD // _BN - 1)
    def _():
        out_ref[...] *= 1.0 / N


def _tc_final(h, g, b):
    return pl.pallas_call(
        _final_body,
        grid=(N_PAD // _BN,),
        in_specs=[
            pl.BlockSpec((_BN, H), lambda i: (i, 0)),
            pl.BlockSpec((1, H), lambda i: (0, 0)),
            pl.BlockSpec((1, H), lambda i: (0, 0)),
        ],
        out_specs=pl.BlockSpec((1, H), lambda i: (0, 0)),
        out_shape=jax.ShapeDtypeStruct((1, H), jnp.float32),
    )(h, g, b)


def kernel(x, edge_index, edge_attr, proj_W, proj_b, msg_W1, msg_b1, msg_W2,
           msg_b2, upd_W, upd_b, ln_g, ln_b, out_g, out_b):
    f32 = jnp.float32
    i_idx = edge_index[0].astype(jnp.int32)
    j_idx = edge_index[1].astype(jnp.int32)
    pad_e = E_PAD - E

    gi = jnp.concatenate([i_idx, jnp.zeros((pad_e,), jnp.int32)])
    gj = jnp.concatenate([j_idx + N_PAD, jnp.full((pad_e,), N_PAD, jnp.int32)])
    gidx = jnp.stack(
        [gi.reshape(NW, IDXROWS, 128), gj.reshape(NW, IDXROWS, 128)], axis=1
    )
    sidx = jnp.concatenate(
        [i_idx, jnp.full((pad_e,), N, jnp.int32)]
    ).reshape(NW, IDXROWS, 128)

    x_pad = jnp.pad(x, ((0, N_PAD - N), (0, 0)))
    ea_pad = jnp.pad(edge_attr, ((0, pad_e), (0, 0)))
    zrow = jnp.zeros((TILE_ROWS, H), f32)

    h = _tc_proj(x_pad, proj_W, proj_b.reshape(1, H))
    for l in range(L):
        w1 = msg_W1[l]
        tbl2 = _tc_ab(h, w1[:H], w1[H : 2 * H], msg_b1[l].reshape(1, H))
        gsum = _sc_gather_kernel()(tbl2.reshape(2 * N_PAD, H), gidx)
        m = _tc_edge(gsum, ea_pad, w1[2 * H :], msg_W2[l], msg_b2[l].reshape(1, H))
        scat = _sc_scatter_kernel()(m, sidx, zrow)
        h = _tc_upd(
            h,
            scat,
            upd_W[l][:H],
            upd_W[l][H:],
            upd_b[l].reshape(1, H),
            ln_g[l].reshape(1, H),
            ln_b[l].reshape(1, H),
        )
    return _tc_final(h, out_g.reshape(1, H), out_b.reshape(1, H))


# speedup vs baseline: 2.1529x; 1.3507x over previous
import functools

import jax
import jax.numpy as jnp
from jax import lax
from jax.experimental import pallas as pl
from jax.experimental.pallas import tpu as pltpu
from jax.experimental.pallas import tpu_sc as plsc

N = 10000
E = 320000
D = 128
ED = 16
H = 128
L = 3

N_PAD = 10240
E_PAD = 327680
NW = 32
EPW = E_PAD // NW
IDXROWS = EPW // 128
CHUNK = 128
NCH = EPW // CHUNK
TILE_ROWS = N_PAD // 16

@functools.lru_cache(maxsize=None)
def _mesh():
    return plsc.VectorSubcoreMesh(
        core_axis_name="c", subcore_axis_name="s", num_cores=2, num_subcores=16
    )


def _gather_body(tbl_hbm, gidx_hbm, out_hbm, idx_v, bufs, sgi, sgj, swb):
    wid = lax.axis_index("s") * 2 + lax.axis_index("c")
    pltpu.sync_copy(gidx_hbm.at[wid], idx_v)
    base = wid * EPW

    def gi_start(c, b):
        pltpu.async_copy(tbl_hbm.at[idx_v.at[0, c]], bufs.at[b], sgi)

    def gi_wait():
        pltpu.make_async_copy(tbl_hbm.at[idx_v.at[0, 0]], bufs.at[0], sgi).wait()

    def gj_start(c, b):
        pltpu.async_copy(tbl_hbm.at[idx_v.at[1, c]], bufs.at[b], sgj, add=True)

    def gj_wait():
        pltpu.make_async_copy(tbl_hbm.at[idx_v.at[1, 0]], bufs.at[0], sgj).wait()

    def wb_start(c, b):
        pltpu.async_copy(bufs.at[b], out_hbm.at[pl.ds(base + c * CHUNK, CHUNK)], swb)

    def wb_wait():
        pltpu.make_async_copy(
            bufs.at[0], out_hbm.at[pl.ds(base, CHUNK)], swb
        ).wait()

    gi_start(0, 0)

    @pl.loop(0, NCH)
    def _chunk(c):
        b = lax.rem(c, 3)
        gi_wait()

        @pl.when(c >= 2)
        def _():
            wb_wait()

        @pl.when(c >= 1)
        def _():
            gj_wait()
            wb_start(c - 1, lax.rem(c + 2, 3))

        gj_start(c, b)

        @pl.when(c <= NCH - 2)
        def _():
            gi_start(c + 1, lax.rem(c + 1, 3))

    wb_wait()
    gj_wait()
    wb_start(NCH - 1, lax.rem(NCH - 1, 3))
    wb_wait()


@functools.lru_cache(maxsize=None)
def _sc_gather_kernel():
    return pl.kernel(
        _gather_body,
        out_type=jax.ShapeDtypeStruct((E_PAD, H), jnp.float32),
        mesh=_mesh(),
        scratch_types=[
            pltpu.VMEM((2, IDXROWS, 128), jnp.int32),
            pltpu.VMEM((3, CHUNK, H), jnp.float32),
            pltpu.SemaphoreType.DMA,
            pltpu.SemaphoreType.DMA,
            pltpu.SemaphoreType.DMA,
        ],
    )


def _scatter_body(m_hbm, sidx_hbm, zrow_hbm, out_hbm, idx_v, bufs, shared, sld, ssc):
    cid = lax.axis_index("c")
    sid = lax.axis_index("s")
    wid = sid * 2 + cid
    pltpu.sync_copy(sidx_hbm.at[wid], idx_v)
    pltpu.sync_copy(zrow_hbm, shared.at[pl.ds(sid * TILE_ROWS, TILE_ROWS)])
    plsc.subcore_barrier()
    base = wid * EPW

    def load_start(c, b):
        pltpu.async_copy(m_hbm.at[pl.ds(base + c * CHUNK, CHUNK)], bufs.at[b], sld)

    def load_wait():
        pltpu.make_async_copy(m_hbm.at[pl.ds(base, CHUNK)], bufs.at[0], sld).wait()

    def scat_start(c, b):
        pltpu.async_copy(bufs.at[b], shared.at[idx_v.at[c]], ssc, add=True)

    def scat_wait():
        pltpu.make_async_copy(bufs.at[0], shared.at[idx_v.at[0]], ssc).wait()

    load_start(0, 0)

    @pl.loop(0, NCH)
    def _chunk(c):
        b = lax.rem(c, 2)
        load_wait()

        @pl.when(c >= 1)
        def _():
            scat_wait()

        scat_start(c, b)

        @pl.when(c <= NCH - 2)
        def _():
            load_start(c + 1, 1 - b)

    scat_wait()
    plsc.subcore_barrier()
    pltpu.sync_copy(
        shared.at[pl.ds(sid * TILE_ROWS, TILE_ROWS)],
        out_hbm.at[cid, pl.ds(sid * TILE_ROWS, TILE_ROWS)],
    )


@functools.lru_cache(maxsize=None)
def _sc_scatter_kernel():
    return pl.kernel(
        _scatter_body,
        out_type=jax.ShapeDtypeStruct((2, N_PAD, H), jnp.float32),
        mesh=_mesh(),
        scratch_types=[
            pltpu.VMEM((IDXROWS, 128), jnp.int32),
            pltpu.VMEM((2, CHUNK, H), jnp.float32),
            pltpu.VMEM_SHARED((N_PAD, H), jnp.float32),
            pltpu.SemaphoreType.DMA,
            pltpu.SemaphoreType.DMA,
        ],
    )


_BN = 1024


def _proj_body(x_ref, w_ref, b_ref, out_ref):
    out_ref[...] = (
        jnp.dot(x_ref[...], w_ref[...], preferred_element_type=jnp.float32)
        + b_ref[...]
    )


def _tc_proj(x, w, b):
    return pl.pallas_call(
        _proj_body,
        grid=(N_PAD // _BN,),
        in_specs=[
            pl.BlockSpec((_BN, D), lambda i: (i, 0)),
            pl.BlockSpec((D, H), lambda i: (0, 0)),
            pl.BlockSpec((1, H), lambda i: (0, 0)),
        ],
        out_specs=pl.BlockSpec((_BN, H), lambda i: (i, 0)),
        out_shape=jax.ShapeDtypeStruct((N_PAD, H), jnp.float32),
    )(x, w, b)


def _ab_body(h_ref, wa_ref, wb_ref, b1_ref, out_ref):
    hb = h_ref[...]
    out_ref[0] = (
        jnp.dot(hb, wa_ref[...], preferred_element_type=jnp.float32) + b1_ref[...]
    )
    out_ref[1] = jnp.dot(hb, wb_ref[...], preferred_element_type=jnp.float32)


def _tc_ab(h, wa, wb, b1):
    return pl.pallas_call(
        _ab_body,
        grid=(N_PAD // _BN,),
        in_specs=[
            pl.BlockSpec((_BN, H), lambda i: (i, 0)),
            pl.BlockSpec((H, H), lambda i: (0, 0)),
            pl.BlockSpec((H, H), lambda i: (0, 0)),
            pl.BlockSpec((1, H), lambda i: (0, 0)),
        ],
        out_specs=pl.BlockSpec((2, _BN, H), lambda i: (0, i, 0)),
        out_shape=jax.ShapeDtypeStruct((2, N_PAD, H), jnp.float32),
    )(h, wa, wb, b1)


def _edge_body(g_ref, ea_ref, wc_ref, w2_ref, b2_ref, out_ref):
    m1 = jnp.maximum(
        g_ref[...]
        + jnp.dot(ea_ref[...], wc_ref[...], preferred_element_type=jnp.float32),
        0.0,
    )
    out_ref[...] = jnp.maximum(
        jnp.dot(m1, w2_ref[...], preferred_element_type=jnp.float32) + b2_ref[...],
        0.0,
    )


def _tc_edge(gsum, ea, wc, w2, b2):
    return pl.pallas_call(
        _edge_body,
        grid=(E_PAD // _BN,),
        in_specs=[
            pl.BlockSpec((_BN, H), lambda i: (i, 0)),
            pl.BlockSpec((_BN, ED), lambda i: (i, 0)),
            pl.BlockSpec((ED, H), lambda i: (0, 0)),
            pl.BlockSpec((H, H), lambda i: (0, 0)),
            pl.BlockSpec((1, H), lambda i: (0, 0)),
        ],
        out_specs=pl.BlockSpec((_BN, H), lambda i: (i, 0)),
        out_shape=jax.ShapeDtypeStruct((E_PAD, H), jnp.float32),
    )(gsum, ea, wc, w2, b2)


def _upd_body(h_ref, a0_ref, a1_ref, wh_ref, wa_ref, bu_ref, g_ref, b_ref, out_ref):
    hb = h_ref[...]
    agg = a0_ref[0] + a1_ref[0]
    o = (
        jnp.dot(hb, wh_ref[...], preferred_element_type=jnp.float32)
        + jnp.dot(agg, wa_ref[...], preferred_element_type=jnp.float32)
        + bu_ref[...]
    )
    o = jnp.maximum(o, 0.0) + hb
    mu = jnp.mean(o, axis=1, keepdims=True)
    var = jnp.mean((o - mu) * (o - mu), axis=1, keepdims=True)
    out_ref[...] = (o - mu) * lax.rsqrt(var + 1e-5) * g_ref[...] + b_ref[...]


def _tc_upd(h, scat, wh, wa, bu, g, b):
    return pl.pallas_call(
        _upd_body,
        grid=(N_PAD // _BN,),
        in_specs=[
            pl.BlockSpec((_BN, H), lambda i: (i, 0)),
            pl.BlockSpec((1, _BN, H), lambda i: (0, i, 0)),
            pl.BlockSpec((1, _BN, H), lambda i: (1, i, 0)),
            pl.BlockSpec((H, H), lambda i: (0, 0)),
            pl.BlockSpec((H, H), lambda i: (0, 0)),
            pl.BlockSpec((1, H), lambda i: (0, 0)),
            pl.BlockSpec((1, H), lambda i: (0, 0)),
            pl.BlockSpec((1, H), lambda i: (0, 0)),
        ],
        out_specs=pl.BlockSpec((_BN, H), lambda i: (i, 0)),
        out_shape=jax.ShapeDtypeStruct((N_PAD, H), jnp.float32),
    )(h, scat, scat, wh, wa, bu, g, b)


def _final_body(h_ref, g_ref, b_ref, out_ref):
    i = pl.program_id(0)
    hb = h_ref[...]
    mu = jnp.mean(hb, axis=1, keepdims=True)
    var = jnp.mean((hb - mu) * (hb - mu), axis=1, keepdims=True)
    y = (hb - mu) * lax.rsqrt(var + 1e-5) * g_ref[...] + b_ref[...]
    rows = i * _BN + lax.broadcasted_iota(jnp.int32, (_BN, 1), 0)
    y = jnp.where(rows < N, y, 0.0)
    part = jnp.sum(y, axis=0, keepdims=True)

    @pl.when(i == 0)
    def _():
        out_ref[...] = jnp.zeros_like(out_ref)

    out_ref[...] += part

    @pl.when(i == N_PAD // _BN - 1)
    def _():
        out_ref[...] *= 1.0 / N


def _tc_final(h, g, b):
    return pl.pallas_call(
        _final_body,
        grid=(N_PAD // _BN,),
        in_specs=[
            pl.BlockSpec((_BN, H), lambda i: (i, 0)),
            pl.BlockSpec((1, H), lambda i: (0, 0)),
            pl.BlockSpec((1, H), lambda i: (0, 0)),
        ],
        out_specs=pl.BlockSpec((1, H), lambda i: (0, 0)),
        out_shape=jax.ShapeDtypeStruct((1, H), jnp.float32),
    )(h, g, b)


def kernel(x, edge_index, edge_attr, proj_W, proj_b, msg_W1, msg_b1, msg_W2,
           msg_b2, upd_W, upd_b, ln_g, ln_b, out_g, out_b):
    f32 = jnp.float32
    i_idx = edge_index[0].astype(jnp.int32)
    j_idx = edge_index[1].astype(jnp.int32)
    pad_e = E_PAD - E

    gi = jnp.concatenate([i_idx, jnp.zeros((pad_e,), jnp.int32)])
    gj = jnp.concatenate([j_idx + N_PAD, jnp.full((pad_e,), N_PAD, jnp.int32)])
    gidx = jnp.stack(
        [gi.reshape(NW, IDXROWS, 128), gj.reshape(NW, IDXROWS, 128)], axis=1
    )
    sidx = jnp.concatenate(
        [i_idx, jnp.full((pad_e,), N, jnp.int32)]
    ).reshape(NW, IDXROWS, 128)

    x_pad = jnp.pad(x, ((0, N_PAD - N), (0, 0)))
    ea_pad = jnp.pad(edge_attr, ((0, pad_e), (0, 0)))
    zrow = jnp.zeros((TILE_ROWS, H), f32)

    h = _tc_proj(x_pad, proj_W, proj_b.reshape(1, H))
    for l in range(L):
        w1 = msg_W1[l]
        tbl2 = _tc_ab(h, w1[:H], w1[H : 2 * H], msg_b1[l].reshape(1, H))
        gsum = _sc_gather_kernel()(tbl2.reshape(2 * N_PAD, H), gidx)
        m = _tc_edge(gsum, ea_pad, w1[2 * H :], msg_W2[l], msg_b2[l].reshape(1, H))
        scat = _sc_scatter_kernel()(m, sidx, zrow)
        h = _tc_upd(
            h,
            scat,
            upd_W[l][:H],
            upd_W[l][H:],
            upd_b[l].reshape(1, H),
            ln_g[l].reshape(1, H),
            ln_b[l].reshape(1, H),
        )
    return _tc_final(h, out_g.reshape(1, H), out_b.reshape(1, H))
